# Initial kernel scaffold; baseline (speedup 1.0000x reference)
import jax
import jax.numpy as jnp
from jax.experimental import pallas as pl


def kernel(z, c, fc_w, fc_gamma, fc_beta, fc_mean, fc_var, up1_w, up1_gamma, up1_beta, up1_mean, up1_var, up2_w, up2_gamma, up2_beta, up2_mean, up2_var, up3_w, up3_gamma, up3_beta, up3_mean, up3_var):
    raise NotImplementedError("write your pallas kernel here")


# speedup vs baseline: 2.0511x; 2.0511x over previous
import functools

import numpy as np
import jax
import jax.numpy as jnp
from jax import lax
from jax.experimental import pallas as pl
from jax.experimental.pallas import tpu as pltpu

_EPS = 1e-5
_NB = 8


def _fold_fc(w, gamma, beta, mean, var):
    s = gamma * lax.rsqrt(var + _EPS)
    w_eff = w.T * s[None, :]
    b_eff = beta - s * mean
    return w_eff, b_eff


def _fold_conv(w, gamma, beta, mean, var):
    s = gamma * lax.rsqrt(var + _EPS)
    wf = w * s[:, None, None, None]
    w9 = jnp.transpose(wf, (2, 3, 0, 1)).reshape(9, w.shape[0], w.shape[1])
    return w9, (beta - s * mean)


def _upmat(h, w):
    h2, w2 = 2 * h, 2 * w
    p = np.arange(h2 * w2)
    src = (p // w2 // 2) * w + (p % w2) // 2
    m = np.zeros((h * w, h2 * w2), dtype=np.float32)
    m[src, p] = 1.0
    return m


def _upmat_bd(h, w, nb):
    return np.kron(np.eye(nb, dtype=np.float32), _upmat(h, w))


def _fc_kernel(x_ref, wv_ref, wg_ref, bv_ref, bg_ref, o_ref):
    x = x_ref[...]
    v = jnp.dot(x, wv_ref[...], preferred_element_type=jnp.float32)
    g = jnp.dot(x, wg_ref[...], preferred_element_type=jnp.float32)
    v = v + bv_ref[...]
    g = g + bg_ref[...]
    o_ref[...] = (v * jax.nn.sigmoid(g)).astype(o_ref.dtype)


def _fc_glu(x, wv, wg, bv, bg):
    B = x.shape[0]
    F = wv.shape[1]
    nt = 8
    ft = F // nt
    return pl.pallas_call(
        _fc_kernel,
        out_shape=jax.ShapeDtypeStruct((B, F), jnp.bfloat16),
        grid=(nt,),
        in_specs=[
            pl.BlockSpec((B, x.shape[1]), lambda i: (0, 0)),
            pl.BlockSpec((wv.shape[0], ft), lambda i: (0, i)),
            pl.BlockSpec((wg.shape[0], ft), lambda i: (0, i)),
            pl.BlockSpec((1, ft), lambda i: (0, i)),
            pl.BlockSpec((1, ft), lambda i: (0, i)),
        ],
        out_specs=pl.BlockSpec((B, ft), lambda i: (0, i)),
        compiler_params=pltpu.CompilerParams(
            dimension_semantics=("parallel",)),
    )(x, wv, wg, bv, bg)


def _conv_glu(x_up, w_ref, b_ref, w2, npix):
    cin, nbpix = x_up.shape
    cout2 = w_ref.shape[1]
    cout = cout2 // 2
    h2 = npix // w2
    lw = int(w2).bit_length() - 1

    lane = lax.broadcasted_iota(jnp.int32, (1, nbpix), 1)
    q = lane & (npix - 1)
    xx = q & (w2 - 1)
    yy = q >> lw

    acc = jnp.zeros((cout2, nbpix), jnp.float32)
    for t in range(9):
        dy = t // 3 - 1
        dx = t % 3 - 1
        off = dy * w2 + dx
        if off > 0:
            sh = jnp.concatenate(
                [x_up[:, off:], jnp.zeros((cin, off), x_up.dtype)], axis=1)
        elif off < 0:
            sh = jnp.concatenate(
                [jnp.zeros((cin, -off), x_up.dtype), x_up[:, :nbpix + off]],
                axis=1)
        else:
            sh = x_up
        if off != 0:
            valid = ((xx + dx >= 0) & (xx + dx < w2) &
                     (yy + dy >= 0) & (yy + dy < h2))
            sh = jnp.where(valid, sh, 0)
        acc = acc + jnp.dot(w_ref[t], sh, preferred_element_type=jnp.float32)

    acc = acc + b_ref[...]
    return acc[:cout] * jax.nn.sigmoid(acc[cout:])


def _net_kernel(x_ref, p1_ref, w1_ref, b1_ref, p2_ref, w2_ref, b2_ref,
                p3_ref, w3_ref, b3_ref, o_ref, *, nb):
    x = x_ref[...]

    xu1 = jnp.dot(x, p1_ref[...],
                  preferred_element_type=jnp.float32).astype(jnp.bfloat16)
    y1 = _conv_glu(xu1, w1_ref, b1_ref, 8, 64).astype(jnp.bfloat16)

    xu2 = jnp.dot(y1, p2_ref[...],
                  preferred_element_type=jnp.float32).astype(jnp.bfloat16)
    y2 = _conv_glu(xu2, w2_ref, b2_ref, 16, 256).astype(jnp.bfloat16)

    p3 = p3_ref[...]
    parts = [
        jnp.dot(y2[:, i * 256:(i + 1) * 256], p3,
                preferred_element_type=jnp.float32).astype(jnp.bfloat16)
        for i in range(nb)
    ]
    xu3 = jnp.concatenate(parts, axis=1)
    y3 = _conv_glu(xu3, w3_ref, b3_ref, 32, 1024)

    for i in range(nb):
        o_ref[i] = y3[:, i * 1024:(i + 1) * 1024]


def _up_chain(x1, p1, w1, b1, p2, w2, b2, p3, w3, b3, B, nb):
    mc = x1.shape[0]
    cout3 = w3.shape[1] // 2
    kfn = functools.partial(_net_kernel, nb=nb)
    out = pl.pallas_call(
        kfn,
        out_shape=jax.ShapeDtypeStruct((B, cout3, 1024), jnp.float32),
        grid=(B // nb,),
        in_specs=[
            pl.BlockSpec((mc, nb * 16), lambda i: (0, i)),
            pl.BlockSpec(p1.shape, lambda i: (0, 0)),
            pl.BlockSpec(w1.shape, lambda i: (0, 0, 0)),
            pl.BlockSpec(b1.shape, lambda i: (0, 0)),
            pl.BlockSpec(p2.shape, lambda i: (0, 0)),
            pl.BlockSpec(w2.shape, lambda i: (0, 0, 0)),
            pl.BlockSpec(b2.shape, lambda i: (0, 0)),
            pl.BlockSpec(p3.shape, lambda i: (0, 0)),
            pl.BlockSpec(w3.shape, lambda i: (0, 0, 0)),
            pl.BlockSpec(b3.shape, lambda i: (0, 0)),
        ],
        out_specs=pl.BlockSpec((nb, cout3, 1024), lambda i: (i, 0, 0)),
        compiler_params=pltpu.CompilerParams(
            dimension_semantics=("parallel",)),
    )(x1, p1, w1, b1, p2, w2, b2, p3, w3, b3)
    return out


def kernel(z, c, fc_w, fc_gamma, fc_beta, fc_mean, fc_var,
           up1_w, up1_gamma, up1_beta, up1_mean, up1_var,
           up2_w, up2_gamma, up2_beta, up2_mean, up2_var,
           up3_w, up3_gamma, up3_beta, up3_mean, up3_var):
    B = z.shape[0]
    nb = _NB
    bf = jnp.bfloat16

    w_eff, b_eff = _fold_fc(fc_w, fc_gamma, fc_beta, fc_mean, fc_var)
    F = w_eff.shape[1] // 2
    mc = F // 16
    wv = w_eff[:, :F].astype(bf)
    wg = w_eff[:, F:].astype(bf)
    bv = b_eff[:F].reshape(1, F).astype(jnp.float32)
    bg = b_eff[F:].reshape(1, F).astype(jnp.float32)
    x_in = jnp.concatenate([c, z], axis=1).astype(bf)

    w1, t1 = _fold_conv(up1_w, up1_gamma, up1_beta, up1_mean, up1_var)
    w2, t2 = _fold_conv(up2_w, up2_gamma, up2_beta, up2_mean, up2_var)
    w3, t3 = _fold_conv(up3_w, up3_gamma, up3_beta, up3_mean, up3_var)
    w1 = w1.astype(bf)
    w2 = w2.astype(bf)
    w3 = w3.astype(bf)
    b1 = t1.reshape(-1, 1).astype(jnp.float32)
    b2 = t2.reshape(-1, 1).astype(jnp.float32)
    b3 = t3.reshape(-1, 1).astype(jnp.float32)

    p1 = jnp.asarray(_upmat_bd(4, 4, nb), bf)
    p2 = jnp.asarray(_upmat_bd(8, 8, nb), bf)
    p3 = jnp.asarray(_upmat(16, 16), bf)

    y = _fc_glu(x_in, wv, wg, bv, bg)

    x1 = y.reshape(B, mc, 16).transpose(1, 0, 2).reshape(mc, B * 16)

    out = _up_chain(x1, p1, w1, b1, p2, w2, b2, p3, w3, b3, B, nb)
    return out.reshape(B, out.shape[1], 32, 32)


# speedup vs baseline: 2.9586x; 1.4425x over previous
import functools

import numpy as np
import jax
import jax.numpy as jnp
from jax import lax
from jax.experimental import pallas as pl
from jax.experimental.pallas import tpu as pltpu

_EPS = 1e-5
_NB = 8

_U = {0: (-1, 0), 1: (0, 1)}
_KTAP = {0: {-1: (0,), 0: (1, 2)}, 1: {0: (0, 1), 1: (2,)}}


def _fold_fc(w, gamma, beta, mean, var):
    s = gamma * lax.rsqrt(var + _EPS)
    w_eff = w * s[:, None]
    b_eff = beta - s * mean
    return w_eff, b_eff


def _fold_parity(w, gamma, beta, mean, var):
    s = gamma * lax.rsqrt(var + _EPS)
    wf = w * s[:, None, None, None]
    planes = []
    for a in (0, 1):
        for b in (0, 1):
            blocks = []
            for u in _U[a]:
                for v in _U[b]:
                    weff = 0.0
                    for ky in _KTAP[a][u]:
                        for kx in _KTAP[b][v]:
                            weff = weff + wf[:, :, ky, kx]
                    blocks.append(weff)
            planes.append(jnp.concatenate(blocks, axis=1))
    return jnp.stack(planes), beta - s * mean


def _scatmat(h, w, a, b):
    i, j = np.mgrid[0:h, 0:w]
    src = (i * w + j).ravel()
    dst = ((2 * i + a) * 2 * w + 2 * j + b).ravel()
    m = np.zeros((h * w, 4 * h * w), dtype=np.float32)
    m[src, dst] = 1.0
    return m


def _scat_bd(h, w, nb):
    eye = np.eye(nb, dtype=np.float32)
    return np.stack([np.kron(eye, _scatmat(h, w, a, b))
                     for a in (0, 1) for b in (0, 1)])


def _scat_cat(h, w):
    return np.concatenate([_scatmat(h, w, a, b)
                           for a in (0, 1) for b in (0, 1)], axis=0)


def _fc_kernel(x_ref, wv_ref, wg_ref, bv_ref, bg_ref, o_ref):
    dn = (((1,), (1,)), ((), ()))
    x = x_ref[...]
    v = lax.dot_general(x, wv_ref[...], dn,
                        preferred_element_type=jnp.float32)
    g = lax.dot_general(x, wg_ref[...], dn,
                        preferred_element_type=jnp.float32)
    v = v + bv_ref[...]
    g = g + bg_ref[...]
    o_ref[...] = (v * jax.nn.sigmoid(g)).astype(o_ref.dtype)


def _fc_glu(x, wv, wg, bv, bg):
    B = x.shape[0]
    F = wv.shape[0]
    nt = 8
    ft = F // nt
    return pl.pallas_call(
        _fc_kernel,
        out_shape=jax.ShapeDtypeStruct((B, F), jnp.bfloat16),
        grid=(nt,),
        in_specs=[
            pl.BlockSpec((B, x.shape[1]), lambda i: (0, 0)),
            pl.BlockSpec((ft, wv.shape[1]), lambda i: (i, 0)),
            pl.BlockSpec((ft, wg.shape[1]), lambda i: (i, 0)),
            pl.BlockSpec((1, ft), lambda i: (0, i)),
            pl.BlockSpec((1, ft), lambda i: (0, i)),
        ],
        out_specs=pl.BlockSpec((B, ft), lambda i: (0, i)),
        compiler_params=pltpu.CompilerParams(
            dimension_semantics=("parallel",)),
    )(x, wv, wg, bv, bg)


def _shifted(x, w_in, npix):
    cin, nbpix = x.shape
    h_in = npix // w_in
    lw = int(w_in).bit_length() - 1
    lane = lax.broadcasted_iota(jnp.int32, (1, nbpix), 1)
    q = lane & (npix - 1)
    xx = q & (w_in - 1)
    yy = q >> lw

    d = {}
    for u in (-1, 0, 1):
        for v in (-1, 0, 1):
            off = u * w_in + v
            if off > 0:
                sh = jnp.concatenate(
                    [x[:, off:], jnp.zeros((cin, off), x.dtype)], axis=1)
            elif off < 0:
                sh = jnp.concatenate(
                    [jnp.zeros((cin, -off), x.dtype), x[:, :nbpix + off]],
                    axis=1)
            else:
                sh = x
            if off != 0:
                valid = ((xx + v >= 0) & (xx + v < w_in) &
                         (yy + u >= 0) & (yy + u < h_in))
                sh = jnp.where(valid, sh, 0)
            d[(u, v)] = sh
    return d


def _parity_planes(x, wp_ref, b_ref, w_in, npix):
    cout2 = wp_ref.shape[1]
    cout = cout2 // 2
    shd = _shifted(x, w_in, npix)
    planes = []
    pl_i = 0
    for a in (0, 1):
        for b in (0, 1):
            sh_all = jnp.concatenate(
                [shd[(u, v)] for u in _U[a] for v in _U[b]], axis=0)
            acc = jnp.dot(wp_ref[pl_i], sh_all,
                          preferred_element_type=jnp.float32)
            acc = acc + b_ref[...]
            y = acc[:cout] * jax.nn.sigmoid(acc[cout:])
            planes.append(y.astype(jnp.bfloat16))
            pl_i += 1
    return planes


def _net_kernel(x_ref, w1_ref, b1_ref, s1_ref, w2_ref, b2_ref, s2_ref,
                w3_ref, b3_ref, s3_ref, o_ref, *, nb):
    x = x_ref[...]

    pl1 = _parity_planes(x, w1_ref, b1_ref, 4, 16)
    y1 = sum(jnp.dot(pl1[i], s1_ref[i], preferred_element_type=jnp.float32)
             for i in range(4)).astype(jnp.bfloat16)

    pl2 = _parity_planes(y1, w2_ref, b2_ref, 8, 64)
    y2 = sum(jnp.dot(pl2[i], s2_ref[i], preferred_element_type=jnp.float32)
             for i in range(4)).astype(jnp.bfloat16)

    pl3 = _parity_planes(y2, w3_ref, b3_ref, 16, 256)
    cout3 = pl3[0].shape[0]
    stacked = [
        jnp.concatenate([p[:, i * 256:(i + 1) * 256] for i in range(nb)],
                        axis=0)
        for p in pl3
    ]
    g = jnp.concatenate(stacked, axis=1)
    out = jnp.dot(g, s3_ref[...], preferred_element_type=jnp.float32)
    o_ref[...] = out.reshape(nb, cout3, 1024)


def _up_chain(x1, w1, b1, s1, w2, b2, s2, w3, b3, s3, B, nb):
    mc = x1.shape[0]
    cout3 = w3.shape[1] // 2
    kfn = functools.partial(_net_kernel, nb=nb)
    out = pl.pallas_call(
        kfn,
        out_shape=jax.ShapeDtypeStruct((B, cout3, 1024), jnp.float32),
        grid=(B // nb,),
        in_specs=[
            pl.BlockSpec((mc, nb * 16), lambda i: (0, i)),
            pl.BlockSpec(w1.shape, lambda i: (0, 0, 0)),
            pl.BlockSpec(b1.shape, lambda i: (0, 0)),
            pl.BlockSpec(s1.shape, lambda i: (0, 0, 0)),
            pl.BlockSpec(w2.shape, lambda i: (0, 0, 0)),
            pl.BlockSpec(b2.shape, lambda i: (0, 0)),
            pl.BlockSpec(s2.shape, lambda i: (0, 0, 0)),
            pl.BlockSpec(w3.shape, lambda i: (0, 0, 0)),
            pl.BlockSpec(b3.shape, lambda i: (0, 0)),
            pl.BlockSpec(s3.shape, lambda i: (0, 0)),
        ],
        out_specs=pl.BlockSpec((nb, cout3, 1024), lambda i: (i, 0, 0)),
        compiler_params=pltpu.CompilerParams(
            dimension_semantics=("parallel",)),
    )(x1, w1, b1, s1, w2, b2, s2, w3, b3, s3)
    return out


def kernel(z, c, fc_w, fc_gamma, fc_beta, fc_mean, fc_var,
           up1_w, up1_gamma, up1_beta, up1_mean, up1_var,
           up2_w, up2_gamma, up2_beta, up2_mean, up2_var,
           up3_w, up3_gamma, up3_beta, up3_mean, up3_var):
    B = z.shape[0]
    nb = _NB
    bf = jnp.bfloat16

    w_eff, b_eff = _fold_fc(fc_w, fc_gamma, fc_beta, fc_mean, fc_var)
    F = w_eff.shape[0] // 2
    mc = F // 16
    wv = w_eff[:F].astype(bf)
    wg = w_eff[F:].astype(bf)
    bv = b_eff[:F].reshape(1, F).astype(jnp.float32)
    bg = b_eff[F:].reshape(1, F).astype(jnp.float32)
    x_in = jnp.concatenate([c, z], axis=1).astype(bf)

    w1, t1 = _fold_parity(up1_w, up1_gamma, up1_beta, up1_mean, up1_var)
    w2, t2 = _fold_parity(up2_w, up2_gamma, up2_beta, up2_mean, up2_var)
    w3, t3 = _fold_parity(up3_w, up3_gamma, up3_beta, up3_mean, up3_var)
    w1 = w1.astype(bf)
    w2 = w2.astype(bf)
    w3 = w3.astype(bf)
    b1 = t1.reshape(-1, 1).astype(jnp.float32)
    b2 = t2.reshape(-1, 1).astype(jnp.float32)
    b3 = t3.reshape(-1, 1).astype(jnp.float32)

    s1 = jnp.asarray(_scat_bd(4, 4, nb), bf)
    s2 = jnp.asarray(_scat_bd(8, 8, nb), bf)
    s3 = jnp.asarray(_scat_cat(16, 16), bf)

    y = _fc_glu(x_in, wv, wg, bv, bg)

    x1 = y.reshape(B, mc, 16).transpose(1, 0, 2).reshape(mc, B * 16)

    out = _up_chain(x1, w1, b1, s1, w2, b2, s2, w3, b3, s3, B, nb)
    return out.reshape(B, out.shape[1], 32, 32)


# speedup vs baseline: 3.7208x; 1.2576x over previous
import functools

import numpy as np
import jax
import jax.numpy as jnp
from jax import lax
from jax.experimental import pallas as pl
from jax.experimental.pallas import tpu as pltpu

_EPS = 1e-5
_NB = 16

_U = {0: (-1, 0), 1: (0, 1)}
_KTAP = {0: {-1: (0,), 0: (1, 2)}, 1: {0: (0, 1), 1: (2,)}}


def _fold_fc(w, gamma, beta, mean, var):
    s = gamma * lax.rsqrt(var + _EPS)
    w_eff = w * s[:, None]
    b_eff = beta - s * mean
    return w_eff, b_eff


def _fold_parity(w, gamma, beta, mean, var):
    s = gamma * lax.rsqrt(var + _EPS)
    wf = w * s[:, None, None, None]
    planes = []
    for a in (0, 1):
        for b in (0, 1):
            blocks = []
            for u in _U[a]:
                for v in _U[b]:
                    weff = 0.0
                    for ky in _KTAP[a][u]:
                        for kx in _KTAP[b][v]:
                            weff = weff + wf[:, :, ky, kx]
                    blocks.append(weff)
            planes.append(jnp.concatenate(blocks, axis=1))
    return jnp.stack(planes), beta - s * mean


def _scatmat(h, w, a, b):
    i, j = np.mgrid[0:h, 0:w]
    src = (i * w + j).ravel()
    dst = ((2 * i + a) * 2 * w + 2 * j + b).ravel()
    m = np.zeros((h * w, 4 * h * w), dtype=np.float32)
    m[src, dst] = 1.0
    return m


def _scat_bd(h, w, nb):
    eye = np.eye(nb, dtype=np.float32)
    return np.stack([np.kron(eye, _scatmat(h, w, a, b))
                     for a in (0, 1) for b in (0, 1)])


def _scat_cat(h, w):
    return np.concatenate([_scatmat(h, w, a, b)
                           for a in (0, 1) for b in (0, 1)], axis=0)


def _fc_kernel(x_ref, wv_ref, wg_ref, bv_ref, bg_ref, o_ref):
    dn = (((1,), (1,)), ((), ()))
    x = x_ref[...]
    v = lax.dot_general(x, wv_ref[...], dn,
                        preferred_element_type=jnp.float32)
    g = lax.dot_general(x, wg_ref[...], dn,
                        preferred_element_type=jnp.float32)
    v = v + bv_ref[...]
    g = g + bg_ref[...]
    o_ref[...] = (v * jax.nn.sigmoid(g)).astype(o_ref.dtype)


def _fc_glu(x, wv, wg, bv, bg):
    B = x.shape[0]
    F = wv.shape[0]
    nt = 8
    ft = F // nt
    return pl.pallas_call(
        _fc_kernel,
        out_shape=jax.ShapeDtypeStruct((B, F), jnp.bfloat16),
        grid=(nt,),
        in_specs=[
            pl.BlockSpec((B, x.shape[1]), lambda i: (0, 0)),
            pl.BlockSpec((ft, wv.shape[1]), lambda i: (i, 0)),
            pl.BlockSpec((ft, wg.shape[1]), lambda i: (i, 0)),
            pl.BlockSpec((1, ft), lambda i: (0, i)),
            pl.BlockSpec((1, ft), lambda i: (0, i)),
        ],
        out_specs=pl.BlockSpec((B, ft), lambda i: (0, i)),
        compiler_params=pltpu.CompilerParams(
            dimension_semantics=("parallel",)),
    )(x, wv, wg, bv, bg)


def _shifted(x, w_in, npix):
    cin, nbpix = x.shape
    h_in = npix // w_in
    lw = int(w_in).bit_length() - 1
    lane = lax.broadcasted_iota(jnp.int32, (1, nbpix), 1)
    q = lane & (npix - 1)
    xx = q & (w_in - 1)
    yy = q >> lw

    d = {}
    for u in (-1, 0, 1):
        for v in (-1, 0, 1):
            off = u * w_in + v
            if off > 0:
                sh = jnp.concatenate(
                    [x[:, off:], jnp.zeros((cin, off), x.dtype)], axis=1)
            elif off < 0:
                sh = jnp.concatenate(
                    [jnp.zeros((cin, -off), x.dtype), x[:, :nbpix + off]],
                    axis=1)
            else:
                sh = x
            if off != 0:
                valid = ((xx + v >= 0) & (xx + v < w_in) &
                         (yy + u >= 0) & (yy + u < h_in))
                sh = jnp.where(valid, sh, 0)
            d[(u, v)] = sh
    return d


def _parity_planes(x, wp_ref, b_ref, w_in, npix):
    cout2 = wp_ref.shape[1]
    cout = cout2 // 2
    shd = _shifted(x, w_in, npix)
    planes = []
    pl_i = 0
    for a in (0, 1):
        for b in (0, 1):
            sh_all = jnp.concatenate(
                [shd[(u, v)] for u in _U[a] for v in _U[b]], axis=0)
            acc = jnp.dot(wp_ref[pl_i], sh_all,
                          preferred_element_type=jnp.float32)
            acc = acc + b_ref[...]
            y = acc[:cout] * jax.nn.sigmoid(acc[cout:])
            planes.append(y.astype(jnp.bfloat16))
            pl_i += 1
    return planes


def _net_kernel(x_ref, w1_ref, b1_ref, s1_ref, w2_ref, b2_ref, s2_ref,
                w3_ref, b3_ref, s3_ref, o_ref, *, nb):
    x = x_ref[...]

    pl1 = _parity_planes(x, w1_ref, b1_ref, 4, 16)
    y1 = sum(jnp.dot(pl1[i], s1_ref[i], preferred_element_type=jnp.float32)
             for i in range(4)).astype(jnp.bfloat16)

    pl2 = _parity_planes(y1, w2_ref, b2_ref, 8, 64)
    c2 = pl2[0].shape[0]
    stacked2 = [
        jnp.concatenate([p[:, i * 64:(i + 1) * 64] for i in range(nb)],
                        axis=0)
        for p in pl2
    ]
    g2 = jnp.concatenate(stacked2, axis=1)
    o2 = jnp.dot(g2, s2_ref[...], preferred_element_type=jnp.float32)
    y2 = jnp.concatenate(
        [o2[i * c2:(i + 1) * c2, :] for i in range(nb)],
        axis=1).astype(jnp.bfloat16)

    pl3 = _parity_planes(y2, w3_ref, b3_ref, 16, 256)
    cout3 = pl3[0].shape[0]
    stacked = [
        jnp.concatenate([p[:, i * 256:(i + 1) * 256] for i in range(nb)],
                        axis=0)
        for p in pl3
    ]
    g = jnp.concatenate(stacked, axis=1)
    out = jnp.dot(g, s3_ref[...], preferred_element_type=jnp.float32)
    o_ref[...] = out.reshape(nb, cout3, 1024)


def _up_chain(x1, w1, b1, s1, w2, b2, s2, w3, b3, s3, B, nb):
    mc = x1.shape[0]
    cout3 = w3.shape[1] // 2
    kfn = functools.partial(_net_kernel, nb=nb)
    out = pl.pallas_call(
        kfn,
        out_shape=jax.ShapeDtypeStruct((B, cout3, 1024), jnp.float32),
        grid=(B // nb,),
        in_specs=[
            pl.BlockSpec((mc, nb * 16), lambda i: (0, i)),
            pl.BlockSpec(w1.shape, lambda i: (0, 0, 0)),
            pl.BlockSpec(b1.shape, lambda i: (0, 0)),
            pl.BlockSpec(s1.shape, lambda i: (0, 0, 0)),
            pl.BlockSpec(w2.shape, lambda i: (0, 0, 0)),
            pl.BlockSpec(b2.shape, lambda i: (0, 0)),
            pl.BlockSpec(s2.shape, lambda i: (0, 0)),
            pl.BlockSpec(w3.shape, lambda i: (0, 0, 0)),
            pl.BlockSpec(b3.shape, lambda i: (0, 0)),
            pl.BlockSpec(s3.shape, lambda i: (0, 0)),
        ],
        out_specs=pl.BlockSpec((nb, cout3, 1024), lambda i: (i, 0, 0)),
        compiler_params=pltpu.CompilerParams(
            dimension_semantics=("parallel",)),
    )(x1, w1, b1, s1, w2, b2, s2, w3, b3, s3)
    return out


def kernel(z, c, fc_w, fc_gamma, fc_beta, fc_mean, fc_var,
           up1_w, up1_gamma, up1_beta, up1_mean, up1_var,
           up2_w, up2_gamma, up2_beta, up2_mean, up2_var,
           up3_w, up3_gamma, up3_beta, up3_mean, up3_var):
    B = z.shape[0]
    nb = _NB
    bf = jnp.bfloat16

    w_eff, b_eff = _fold_fc(fc_w, fc_gamma, fc_beta, fc_mean, fc_var)
    F = w_eff.shape[0] // 2
    mc = F // 16
    wv = w_eff[:F].astype(bf)
    wg = w_eff[F:].astype(bf)
    bv = b_eff[:F].reshape(1, F).astype(jnp.float32)
    bg = b_eff[F:].reshape(1, F).astype(jnp.float32)
    x_in = jnp.concatenate([c, z], axis=1).astype(bf)

    w1, t1 = _fold_parity(up1_w, up1_gamma, up1_beta, up1_mean, up1_var)
    w2, t2 = _fold_parity(up2_w, up2_gamma, up2_beta, up2_mean, up2_var)
    w3, t3 = _fold_parity(up3_w, up3_gamma, up3_beta, up3_mean, up3_var)
    w1 = w1.astype(bf)
    w2 = w2.astype(bf)
    w3 = w3.astype(bf)
    b1 = t1.reshape(-1, 1).astype(jnp.float32)
    b2 = t2.reshape(-1, 1).astype(jnp.float32)
    b3 = t3.reshape(-1, 1).astype(jnp.float32)

    s1 = jnp.asarray(_scat_bd(4, 4, nb), bf)
    s2 = jnp.asarray(_scat_cat(8, 8), bf)
    s3 = jnp.asarray(_scat_cat(16, 16), bf)

    y = _fc_glu(x_in, wv, wg, bv, bg)

    x1 = y.reshape(B, mc, 16).transpose(1, 0, 2).reshape(mc, B * 16)

    out = _up_chain(x1, w1, b1, s1, w2, b2, s2, w3, b3, s3, B, nb)
    return out.reshape(B, out.shape[1], 32, 32)
